# baseline (speedup 1.0000x reference)
import functools

import jax
import jax.numpy as jnp
from jax import lax
from jax.experimental import pallas as pl
from jax.experimental.pallas import tpu as pltpu
from jax.experimental.pallas import tpu_sc as plsc

ROW_W = 128
NC = 2
NS = 16
NW = NC * NS
L = 16


def _emb_body(idx_hbm, table_hbm, out_hbm, idx_v, rows0, rows1, xv0, xv1,
              sg0, sg1, so0, so1):
    wid = lax.axis_index("s") * NC + lax.axis_index("c")
    per_w = idx_hbm.shape[0] // NW
    d = table_hbm.shape[1]
    n_bt = 128
    rows = (rows0, rows1)
    xv = (xv0, xv1)
    sg = (sg0, sg1)
    so = (so0, so1)
    iota = lax.iota(jnp.int32, L)

    def fire(u, p):
        pltpu.async_copy(table_hbm.at[idx_v.at[u]], rows[p], sg[p])

    def drain_gather(u, p):
        pltpu.make_async_copy(table_hbm.at[idx_v.at[u]], rows[p], sg[p]).wait()

    def transpose_unit(p):
        for l in range(ROW_W // L):
            rowids = iota + (l * L)
            for j in range(d):
                jv = lax.full((L,), j, jnp.int32)
                xv[p][j, pl.ds(l * L, L)] = plsc.load_gather(
                    rows[p], [rowids, jv]
                )

    def start_writeback(u, p):
        g = wid * per_w + u
        h = g // n_bt
        bt = g % n_bt
        for jt in range(d // 8):
            pltpu.async_copy(
                xv[p].at[pl.ds(jt * 8, 8)],
                out_hbm.at[h * 4 * n_bt + jt * n_bt + bt],
                so[p],
            )

    def wait_writeback(p):
        for jt in range(d // 8):
            pltpu.make_async_copy(
                xv[p].at[pl.ds(jt * 8, 8)], out_hbm.at[0], so[p]
            ).wait()

    pltpu.sync_copy(idx_hbm.at[pl.ds(wid * per_w, per_w)], idx_v)
    fire(0, 0)

    def body(uu, carry):
        for p in (0, 1):
            u = 2 * uu + p

            @pl.when(u + 1 < per_w)
            def _():
                fire(u + 1, 1 - p)

            drain_gather(u, p)

            @pl.when(u >= 2)
            def _():
                wait_writeback(p)

            transpose_unit(p)
            start_writeback(u, p)
        return carry

    lax.fori_loop(0, per_w // 2, body, 0)
    wait_writeback(0)
    wait_writeback(1)


def kernel(token_index, table):
    b, h = token_index.shape
    v, d = table.shape
    n = b * h
    idx = token_index.T.reshape(n // ROW_W, ROW_W)
    n_bt = b // ROW_W

    mesh = plsc.VectorSubcoreMesh(core_axis_name="c", subcore_axis_name="s")
    fn = functools.partial(
        pl.kernel,
        mesh=mesh,
        out_type=jax.ShapeDtypeStruct((n * d // (8 * ROW_W), 8, ROW_W),
                                      jnp.float32),
        scratch_types=[
            pltpu.VMEM((n // (ROW_W * NW), ROW_W), jnp.int32),
            pltpu.VMEM((ROW_W, d), jnp.float32),
            pltpu.VMEM((ROW_W, d), jnp.float32),
            pltpu.VMEM((d, ROW_W), jnp.float32),
            pltpu.VMEM((d, ROW_W), jnp.float32),
            pltpu.SemaphoreType.DMA,
            pltpu.SemaphoreType.DMA,
            pltpu.SemaphoreType.DMA,
            pltpu.SemaphoreType.DMA,
        ],
        compiler_params=pltpu.CompilerParams(
            use_tc_tiling_on_sc=False, needs_layout_passes=False
        ),
    )(_emb_body)
    out = fn(idx, table)
    return (
        out.reshape(h, d // 8, n_bt, 8, ROW_W)
        .transpose(2, 4, 0, 1, 3)
        .reshape(b, h, d)
    )


# speedup vs baseline: 1.2623x; 1.2623x over previous
import functools

import jax
import jax.numpy as jnp
from jax import lax
from jax.experimental import pallas as pl
from jax.experimental.pallas import tpu as pltpu
from jax.experimental.pallas import tpu_sc as plsc

ROW_W = 128
K = 10
NC = 2
NS = 16
NW = NC * NS


def _emb_body(idx_hbm, table_hbm, out_hbm, idx_v, rows_v, sg0, sg1, so0, so1):
    wid = lax.axis_index("s") * NC + lax.axis_index("c")
    rows_total = idx_hbm.shape[0]
    per_w = rows_total // NW
    n_g = per_w // K
    base_row = wid * per_w
    d = table_hbm.shape[1]

    sg = (sg0, sg1)
    so = (so0, so1)

    def fire(g, p):
        for j in range(K):
            pltpu.async_copy(
                table_hbm.at[idx_v.at[g * K + j]],
                rows_v.at[p].at[j],
                sg[p],
            )

    def drain_gathers(p):
        pltpu.make_async_copy(out_hbm.at[pl.ds(0, K)], rows_v.at[p], sg[p]).wait()

    def start_writeback(g, p):
        pltpu.async_copy(rows_v.at[p], out_hbm.at[pl.ds(base_row + g * K, K)], so[p])

    def wait_writeback(p):
        pltpu.make_async_copy(rows_v.at[p], out_hbm.at[pl.ds(0, K)], so[p]).wait()

    pltpu.sync_copy(idx_hbm.at[pl.ds(base_row, per_w)], idx_v)
    fire(0, 0)

    def body(gg, carry):
        for p in (0, 1):
            g = 2 * gg + p
            @pl.when(g >= 1)
            def _():
                wait_writeback(1 - p)

            @pl.when(g + 1 < n_g)
            def _():
                fire(g + 1, 1 - p)

            drain_gathers(p)
            start_writeback(g, p)
        return carry

    lax.fori_loop(0, n_g // 2, body, 0)
    wait_writeback((n_g - 1) % 2)


def kernel(token_index, table):
    b, h = token_index.shape
    v, d = table.shape
    n = b * h
    idx = token_index.T.reshape(n // ROW_W, ROW_W)
    per_w = (n // ROW_W) // NW

    mesh = plsc.VectorSubcoreMesh(core_axis_name="c", subcore_axis_name="s")
    fn = functools.partial(
        pl.kernel,
        mesh=mesh,
        out_type=jax.ShapeDtypeStruct((n // ROW_W, ROW_W, d), jnp.float32),
        scratch_types=[
            pltpu.VMEM((per_w, ROW_W), jnp.int32),
            pltpu.VMEM((2, K, ROW_W, d), jnp.float32),
            pltpu.SemaphoreType.DMA,
            pltpu.SemaphoreType.DMA,
            pltpu.SemaphoreType.DMA,
            pltpu.SemaphoreType.DMA,
        ],
        compiler_params=pltpu.CompilerParams(use_tc_tiling_on_sc=False),
    )(_emb_body)
    out = fn(idx, table)
    bt = b // ROW_W
    return out.reshape(h, bt, ROW_W, d).transpose(1, 2, 0, 3).reshape(b, h, d)
